# Initial kernel scaffold; baseline (speedup 1.0000x reference)
import jax
import jax.numpy as jnp
from jax.experimental import pallas as pl


def kernel(node_names, x, edge_attr, edge_index, params):
    raise NotImplementedError("write your pallas kernel here")


# speedup vs baseline: 1.7075x; 1.7075x over previous
import jax
import jax.numpy as jnp
from jax import lax
from jax.experimental import pallas as pl
from jax.experimental.pallas import tpu as pltpu
from jax.experimental.pallas import tpu_sc as plsc

N = 10000
EMB = 128
XINC = 64
D = EMB + XINC
H = 4
DOUT = 128
V = 64
E = 320000
CHUNK = 32000
NC = 2
NS = 16
NW = NC * NS
EPW = E // NW

f32 = jnp.float32
i32 = jnp.int32

BQ = 400
BN = 1000
CL = 400
CM = 40

_SCALE = float(D) ** -0.5


def _front_body(names_ref, x_ref, emb_ref, wqkv_ref, bqkv_ref,
                q_ref, k_ref, v_ref, ei_ref):
    ids = names_ref[...]
    iot = lax.broadcasted_iota(i32, (BQ, V), 1)
    oh = (ids == iot).astype(f32)
    ei = jnp.dot(oh, emb_ref[...], preferred_element_type=f32)
    ei_ref[...] = ei
    xb = jnp.broadcast_to(x_ref[...], (BQ, XINC))
    z = jnp.concatenate([xb, ei], axis=1)
    qkv = jnp.dot(z, wqkv_ref[...], preferred_element_type=f32) + bqkv_ref[...]
    q_ref[...] = qkv[:, :D]
    k_ref[...] = qkv[:, D:2 * D]
    v_ref[...] = qkv[:, 2 * D:]


def _front(names2, x, embed, wqkvT, bqkv2):
    nb = N // BQ
    return pl.pallas_call(
        _front_body,
        grid=(nb,),
        in_specs=[
            pl.BlockSpec((BQ, 1), lambda i: (i, 0)),
            pl.BlockSpec((BQ, 1), lambda i: (i, 0)),
            pl.BlockSpec((V, EMB), lambda i: (0, 0)),
            pl.BlockSpec((D, 3 * D), lambda i: (0, 0)),
            pl.BlockSpec((1, 3 * D), lambda i: (0, 0)),
        ],
        out_specs=[pl.BlockSpec((BQ, D), lambda i: (i, 0))] * 3
        + [pl.BlockSpec((BQ, EMB), lambda i: (i, 0))],
        out_shape=[jax.ShapeDtypeStruct((N, D), f32)] * 3
        + [jax.ShapeDtypeStruct((N, EMB), f32)],
    )(names2, x, embed, wqkvT, bqkv2)


def _att_body(q_ref, k_ref, v_ref, ei_ref, wp_ref, bp_ref, wq_ref, bq_ref,
              g_ref, b_ref, out_ref):
    qb = q_ref[...]
    s = lax.dot_general(qb, k_ref[...], (((1,), (1,)), ((), ())),
                        preferred_element_type=f32) * _SCALE
    mx = jnp.max(s, axis=1, keepdims=True)
    p = jnp.exp(s - mx)
    p = p / jnp.sum(p, axis=1, keepdims=True)
    o = jnp.dot(p, v_ref[...], preferred_element_type=f32)
    o = jnp.dot(o, wp_ref[...], preferred_element_type=f32) + bp_ref[...]
    zz = jnp.dot(o, wq_ref[...], preferred_element_type=f32) + bq_ref[...]
    m = jnp.mean(zz, axis=1, keepdims=True)
    zc = zz - m
    var = jnp.mean(zc * zc, axis=1, keepdims=True)
    ln = zc * lax.rsqrt(var + 1e-5) * g_ref[...] + b_ref[...]
    out_ref[...] = ln * jax.nn.sigmoid(ln) + ei_ref[...]


def _attention(q, k, v, ei, wpT, bp2, wqT, bq2, g2, b2):
    nb = N // BQ
    return pl.pallas_call(
        _att_body,
        grid=(nb,),
        in_specs=[
            pl.BlockSpec((BQ, D), lambda i: (i, 0)),
            pl.BlockSpec((N, D), lambda i: (0, 0)),
            pl.BlockSpec((N, D), lambda i: (0, 0)),
            pl.BlockSpec((BQ, EMB), lambda i: (i, 0)),
            pl.BlockSpec((D, D), lambda i: (0, 0)),
            pl.BlockSpec((1, D), lambda i: (0, 0)),
            pl.BlockSpec((D, EMB), lambda i: (0, 0)),
            pl.BlockSpec((1, EMB), lambda i: (0, 0)),
            pl.BlockSpec((1, EMB), lambda i: (0, 0)),
            pl.BlockSpec((1, EMB), lambda i: (0, 0)),
        ],
        out_specs=pl.BlockSpec((BQ, EMB), lambda i: (i, 0)),
        out_shape=jax.ShapeDtypeStruct((N, EMB), f32),
    )(q, k, v, ei, wpT, bp2, wqT, bq2, g2, b2)


def _head_sums(W, attn_ref):
    cols = [jnp.sum(W[:, h * DOUT:(h + 1) * DOUT] * attn_ref[h:h + 1, :],
                    axis=1, keepdims=True) for h in range(H)]
    return jnp.concatenate(cols, axis=1)


def _pre1_body(x_ref, wi_ref, bi_ref, wj_ref, bj_ref, wt_ref, bt_ref,
               attn_ref, ai_ref, aj_ref, t_ref):
    xb = x_ref[...]
    Wi = jnp.dot(xb, wi_ref[...], preferred_element_type=f32) + bi_ref[...]
    ai_ref[...] = _head_sums(Wi, attn_ref)
    Wj = jnp.dot(xb, wj_ref[...], preferred_element_type=f32) + bj_ref[...]
    aj_ref[...] = _head_sums(Wj, attn_ref)
    t_ref[...] = jnp.dot(xb, wt_ref[...], preferred_element_type=f32) + bt_ref[...]


def _pre2_body(p0_ref, p1_ref, wi_ref, bi_ref, wj_ref, bj_ref, wt_ref, bt_ref,
               attn_ref, ws_ref, bs_ref, ai_ref, aj_ref, t_ref, skip_ref):
    xb = p0_ref[...] + p1_ref[...]
    Wi = jnp.dot(xb, wi_ref[...], preferred_element_type=f32) + bi_ref[...]
    ai_ref[...] = _head_sums(Wi, attn_ref)
    Wj = jnp.dot(xb, wj_ref[...], preferred_element_type=f32) + bj_ref[...]
    aj_ref[...] = _head_sums(Wj, attn_ref)
    t_ref[...] = jnp.dot(xb, wt_ref[...], preferred_element_type=f32) + bt_ref[...]
    skip_ref[...] = jnp.dot(xb, ws_ref[...], preferred_element_type=f32) + bs_ref[...]


def _wspecs():
    return [
        pl.BlockSpec((EMB, H * DOUT), lambda i: (0, 0)),
        pl.BlockSpec((1, H * DOUT), lambda i: (0, 0)),
        pl.BlockSpec((EMB, H * DOUT), lambda i: (0, 0)),
        pl.BlockSpec((1, H * DOUT), lambda i: (0, 0)),
        pl.BlockSpec((EMB, H * DOUT), lambda i: (0, 0)),
        pl.BlockSpec((1, H * DOUT), lambda i: (0, 0)),
        pl.BlockSpec((H, DOUT), lambda i: (0, 0)),
    ]


def _node_pre1(emb, wiT, bi2, wjT, bj2, wtT, bt2, attn):
    nb = N // BN
    return pl.pallas_call(
        _pre1_body,
        grid=(nb,),
        in_specs=[pl.BlockSpec((BN, EMB), lambda i: (i, 0))] + _wspecs(),
        out_specs=[pl.BlockSpec((BN, H), lambda i: (i, 0))] * 2
        + [pl.BlockSpec((BN, H * DOUT), lambda i: (i, 0))],
        out_shape=[jax.ShapeDtypeStruct((N, H), f32)] * 2
        + [jax.ShapeDtypeStruct((N, H * DOUT), f32)],
    )(emb, wiT, bi2, wjT, bj2, wtT, bt2, attn)


def _node_pre2(p0, p1, wiT, bi2, wjT, bj2, wtT, bt2, attn, wsT, bs2):
    nb = N // BN
    return pl.pallas_call(
        _pre2_body,
        grid=(nb,),
        in_specs=[pl.BlockSpec((BN, EMB), lambda i: (i, 0))] * 2 + _wspecs()
        + [pl.BlockSpec((DOUT, EMB), lambda i: (0, 0)),
           pl.BlockSpec((1, EMB), lambda i: (0, 0))],
        out_specs=[pl.BlockSpec((BN, H), lambda i: (i, 0))] * 2
        + [pl.BlockSpec((BN, H * DOUT), lambda i: (i, 0)),
           pl.BlockSpec((BN, EMB), lambda i: (i, 0))],
        out_shape=[jax.ShapeDtypeStruct((N, H), f32)] * 2
        + [jax.ShapeDtypeStruct((N, H * DOUT), f32),
           jax.ShapeDtypeStruct((N, EMB), f32)],
    )(p0, p1, wiT, bi2, wjT, bj2, wtT, bt2, attn, wsT, bs2)


def _logits_sc(ai_flat, aj_flat, src, dst, edge_attr):
    mesh = plsc.VectorSubcoreMesh(core_axis_name="c", subcore_axis_name="s",
                                  num_cores=NC, num_subcores=NS)

    @pl.kernel(
        out_type=jax.ShapeDtypeStruct((H * E,), f32),
        mesh=mesh,
        compiler_params=pltpu.CompilerParams(needs_layout_passes=False),
        scratch_types=[
            pltpu.VMEM((N * H,), f32),
            pltpu.VMEM((N * H,), f32),
            pltpu.VMEM((CL,), i32),
            pltpu.VMEM((CL,), i32),
            pltpu.VMEM((CL,), f32),
            pltpu.VMEM((CL,), f32),
            pltpu.VMEM((CL,), f32),
            pltpu.VMEM((CL,), f32),
            pltpu.VMEM((CL,), f32),
        ],
    )
    def body(ai_hbm, aj_hbm, src_hbm, dst_hbm, ea_hbm, out_hbm,
             ai_v, aj_v, src_v, dst_v, ea_v, o0, o1, o2, o3):
        o_bufs = [o0, o1, o2, o3]
        wid = lax.axis_index("s") * NC + lax.axis_index("c")
        pltpu.sync_copy(ai_hbm, ai_v)
        pltpu.sync_copy(aj_hbm, aj_v)
        nchunks = EPW // CL

        def chunk(ic, _):
            base = wid * EPW + ic * CL
            pltpu.sync_copy(src_hbm.at[pl.ds(base, CL)], src_v)
            pltpu.sync_copy(dst_hbm.at[pl.ds(base, CL)], dst_v)
            pltpu.sync_copy(ea_hbm.at[pl.ds(base, CL)], ea_v)

            def grp(j, _):
                sl = pl.ds(j * 16, 16)
                dv = dst_v[sl] * H
                sv = src_v[sl] * H
                eav = ea_v[sl]
                for h in range(H):
                    hv = jnp.full((16,), h, i32)
                    aih = plsc.load_gather(ai_v, [dv + hv])
                    ajh = plsc.load_gather(aj_v, [sv + hv])
                    o_bufs[h][sl] = (aih + ajh) * eav
                return 0

            lax.fori_loop(0, CL // 16, grp, 0)
            for h in range(H):
                pltpu.sync_copy(o_bufs[h],
                                out_hbm.at[pl.ds(h * E + base, CL)])
            return 0

        lax.fori_loop(0, nchunks, chunk, 0)

    return body(ai_flat, aj_flat, src, dst, edge_attr)


def _softmax_body(x_ref, o_ref):
    s = x_ref[...]
    mx = jnp.max(s, axis=1, keepdims=True)
    p = jnp.exp(s - mx)
    o_ref[...] = p / jnp.sum(p, axis=1, keepdims=True)


def _chunk_softmax(logits_flat):
    nch = E // CHUNK
    rows = H * nch
    br = 8
    lg2 = logits_flat.reshape(rows, CHUNK)
    ap2 = pl.pallas_call(
        _softmax_body,
        grid=(rows // br,),
        in_specs=[pl.BlockSpec((br, CHUNK), lambda i: (i, 0))],
        out_specs=pl.BlockSpec((br, CHUNK), lambda i: (i, 0)),
        out_shape=jax.ShapeDtypeStruct((rows, CHUNK), f32),
    )(lg2)
    return ap2.reshape(H * E)


NPAD = 10240
STR = NPAD // NS
ZR = 32


def _message_sc(src, dst, ap_flat, T):
    mesh = plsc.VectorSubcoreMesh(core_axis_name="c", subcore_axis_name="s",
                                  num_cores=NC, num_subcores=NS)

    @pl.kernel(
        out_type=jax.ShapeDtypeStruct((NC, NPAD, EMB), f32),
        mesh=mesh,
        compiler_params=pltpu.CompilerParams(needs_layout_passes=False),
        scratch_types=[
            pltpu.VMEM((CM,), i32),
            pltpu.VMEM((CM,), i32),
            pltpu.VMEM((CM,), f32),
            pltpu.VMEM((CM,), f32),
            pltpu.VMEM((CM,), f32),
            pltpu.VMEM((CM,), f32),
            pltpu.VMEM((CM, H * DOUT), f32),
            pltpu.VMEM((CM, EMB), f32),
            pltpu.VMEM((ZR, EMB), f32),
            pltpu.VMEM_SHARED((NPAD, EMB), f32),
            pltpu.SemaphoreType.DMA,
        ],
    )
    def body(src_hbm, dst_hbm, ap_hbm, t_hbm, out_hbm,
             src_v, dst_v, ap0, ap1, ap2, ap3, rows_v, m_v, z_v, accum, sem):
        ap_bufs = [ap0, ap1, ap2, ap3]
        cid = lax.axis_index("c")
        sid = lax.axis_index("s")
        wid = sid * NC + cid

        zeros = jnp.zeros((16,), f32)

        def zrow(r, _):
            for g in range(EMB // 16):
                z_v[r, pl.ds(g * 16, 16)] = zeros
            return 0

        lax.fori_loop(0, ZR, zrow, 0)
        for t in range(STR // ZR):
            pltpu.sync_copy(z_v, accum.at[pl.ds(sid * STR + t * ZR, ZR)])
        plsc.subcore_barrier()

        nchunks = EPW // CM

        def chunk(ic, _):
            base = wid * EPW + ic * CM
            pltpu.sync_copy(src_hbm.at[pl.ds(base, CM)], src_v)
            pltpu.sync_copy(dst_hbm.at[pl.ds(base, CM)], dst_v)
            for h in range(H):
                pltpu.sync_copy(ap_hbm.at[pl.ds(h * E + base, CM)],
                                ap_bufs[h])
            pltpu.async_copy(t_hbm.at[src_v], rows_v, sem).wait()

            def edge(e, _):
                ev = jnp.broadcast_to(e, (16,))
                cf = [plsc.load_gather(ap_bufs[h], [ev]) for h in range(H)]
                for g in range(EMB // 16):
                    acc = cf[0] * rows_v[e, pl.ds(g * 16, 16)]
                    for h in range(1, H):
                        acc = acc + cf[h] * rows_v[e, pl.ds(h * DOUT + g * 16, 16)]
                    m_v[e, pl.ds(g * 16, 16)] = acc
                return 0

            lax.fori_loop(0, CM, edge, 0)
            pltpu.sync_copy(m_v, accum.at[dst_v], add=True)
            return 0

        lax.fori_loop(0, nchunks, chunk, 0)
        plsc.subcore_barrier()
        pltpu.sync_copy(accum.at[pl.ds(sid * STR, STR)],
                        out_hbm.at[cid, pl.ds(sid * STR, STR)])

    return body(src, dst, ap_flat, T)


def _final_body(p0_ref, p1_ref, skip1_ref, ws_ref, bs_ref, g_ref, b_ref,
                out_ref):
    e2 = p0_ref[...] + p1_ref[...]
    skip2 = jnp.dot(e2, ws_ref[...], preferred_element_type=f32) + bs_ref[...]
    r = e2 + 0.5 * (skip1_ref[...] + skip2)
    m = jnp.mean(r, axis=1, keepdims=True)
    rc = r - m
    var = jnp.mean(rc * rc, axis=1, keepdims=True)
    out_ref[...] = rc * lax.rsqrt(var + 1e-5) * g_ref[...] + b_ref[...]


def _final(p0, p1, skip1, wsT, bs2, g2, b2):
    nb = N // BN
    return pl.pallas_call(
        _final_body,
        grid=(nb,),
        in_specs=[pl.BlockSpec((BN, EMB), lambda i: (i, 0))] * 3
        + [pl.BlockSpec((EMB, EMB), lambda i: (0, 0)),
           pl.BlockSpec((1, EMB), lambda i: (0, 0)),
           pl.BlockSpec((1, EMB), lambda i: (0, 0)),
           pl.BlockSpec((1, EMB), lambda i: (0, 0))],
        out_specs=pl.BlockSpec((BN, EMB), lambda i: (i, 0)),
        out_shape=jax.ShapeDtypeStruct((N, EMB), f32),
    )(p0, p1, skip1, wsT, bs2, g2, b2)


def kernel(node_names, x, edge_attr, edge_index, params):
    p = params
    names2 = node_names.reshape(N, 1).astype(i32)
    src = edge_index[0]
    dst = edge_index[1]

    q, k, v, ei = _front(
        names2, x, p['embed'], p['qkv_w'].T, p['qkv_b'].reshape(1, -1))
    emb0 = _attention(
        q, k, v, ei, p['proj_w'].T, p['proj_b'].reshape(1, -1),
        p['qlin_w'].T, p['qlin_b'].reshape(1, -1),
        p['qln_g'].reshape(1, -1), p['qln_b'].reshape(1, -1))

    l1, l2 = p['layers'][0], p['layers'][1]
    s1, s2 = p['skips'][0], p['skips'][1]

    ai1, aj1, T1 = _node_pre1(
        emb0, l1['wi_w'].T, l1['wi_b'].reshape(1, -1),
        l1['wj_w'].T, l1['wj_b'].reshape(1, -1),
        l1['tr_w'].T, l1['tr_b'].reshape(1, -1), l1['attn'])
    lg1 = _logits_sc(ai1.reshape(N * H), aj1.reshape(N * H), src, dst,
                     edge_attr)
    ap1 = _chunk_softmax(lg1)
    part1 = _message_sc(src, dst, ap1, T1)

    ai2, aj2, T2, skip1 = _node_pre2(
        part1[0, :N], part1[1, :N], l2['wi_w'].T, l2['wi_b'].reshape(1, -1),
        l2['wj_w'].T, l2['wj_b'].reshape(1, -1),
        l2['tr_w'].T, l2['tr_b'].reshape(1, -1), l2['attn'],
        s1['w'].T, s1['b'].reshape(1, -1))
    lg2 = _logits_sc(ai2.reshape(N * H), aj2.reshape(N * H), src, dst,
                     edge_attr)
    ap2 = _chunk_softmax(lg2)
    part2 = _message_sc(src, dst, ap2, T2)

    emb = _final(part2[0, :N], part2[1, :N], skip1, s2['w'].T,
                 s2['b'].reshape(1, -1), p['norm_g'].reshape(1, -1),
                 p['norm_b'].reshape(1, -1))
    return emb, p['embed'], node_names
